# baseline (speedup 1.0000x reference)
import functools

import jax
import jax.numpy as jnp
from jax import lax
from jax.experimental import pallas as pl
from jax.experimental.pallas import tpu as pltpu
from jax.experimental.pallas import tpu_sc as plsc

F = 26
V = 100000
D = 16
B = 16384
ADAPT = 64
TEMP = 5.0
FD = F * D
BS = B // 128

_NC = 2
_NS = 16
_NW = _NC * _NS
_BC = 1024
_NCH = B // _BC
_UPW = F * _NCH // _NW
_RPW = FD // _NW

_SC_PARAMS = pltpu.CompilerParams(needs_layout_passes=False)
_GC = 512


@functools.cache
def _perm3():
    def build():
        u = jax.random.uniform(jax.random.key(1), (FD, B))
        p = jnp.argsort(u, axis=1).astype(jnp.int32)
        return p.reshape(FD, BS, 128)

    try:
        with jax.ensure_compile_time_eval():
            return build()
    except Exception:
        return build()


_A0_CH = 1000
_A0_N = F * (V // _A0_CH)
_A0_T = (_A0_N + _NW - 1) // _NW


def _sc_relayout(tables3):
    mesh = plsc.VectorSubcoreMesh(core_axis_name="c", subcore_axis_name="s")

    @functools.partial(
        pl.kernel,
        out_type=jax.ShapeDtypeStruct((F * V, D), jnp.float32),
        mesh=mesh,
        scratch_types=[
            pltpu.VMEM((_A0_CH, D), jnp.float32),
        ],
        compiler_params=_SC_PARAMS,
    )
    def k(tab_hbm, out_hbm, in_v):
        wid = lax.axis_index("s") * _NC + lax.axis_index("c")

        def body(t, _):
            u = t * _NW + wid

            @pl.when(u < _A0_N)
            def _():
                f = u // (V // _A0_CH)
                kk = u % (V // _A0_CH)
                pltpu.sync_copy(
                    tab_hbm.at[f, pl.ds(kk * _A0_CH, _A0_CH), :], in_v
                )
                pltpu.sync_copy(
                    in_v,
                    out_hbm.at[pl.ds(f * V + kk * _A0_CH, _A0_CH), :],
                )

            return ()

        lax.fori_loop(0, _A0_T, body, ())

    return k(tables3)


def _sc_gather_t(inputs_flat, tables8):
    mesh = plsc.VectorSubcoreMesh(core_axis_name="c", subcore_axis_name="s")

    @functools.partial(
        pl.kernel,
        out_type=jax.ShapeDtypeStruct((FD, BS, 128), jnp.float32),
        mesh=mesh,
        scratch_types=[
            pltpu.VMEM((_GC,), jnp.int32),
            pltpu.VMEM((_GC,), jnp.int32),
            pltpu.VMEM((_GC,), jnp.int32),
            pltpu.VMEM((_GC, 128), jnp.float32),
            pltpu.VMEM((D * 8, 128), jnp.float32),
            pltpu.SemaphoreType.DMA,
        ],
        compiler_params=_SC_PARAMS,
    )
    def k(inp_hbm, tab_hbm, xt_hbm, raw_v, hi_v, lo_v, rows_v, xt_v, sem):
        wid = lax.axis_index("s") * _NC + lax.axis_index("c")
        iota16 = lax.iota(jnp.int32, 16)

        def unit_body(t, _):
            u = wid * _UPW + t
            f = u // _NCH
            c = u % _NCH
            for sub in range(_BC // _GC):
                base = f * B + c * _BC + sub * _GC
                pltpu.sync_copy(inp_hbm.at[pl.ds(base, _GC)], raw_v)
                off = jnp.full((16,), 0, jnp.int32) + f * (V // 8)

                def idx_body(i, _):
                    raw = raw_v[pl.ds(i * 16, 16)]
                    hi_v[pl.ds(i * 16, 16)] = (
                        lax.shift_right_logical(raw, 3) + off
                    )
                    lo_v[pl.ds(i * 16, 16)] = lax.shift_left(
                        lax.bitwise_and(raw, 7), 4
                    )
                    return ()

                lax.fori_loop(0, _GC // 16, idx_body, (), unroll=4)
                pltpu.async_copy(tab_hbm.at[hi_v], rows_v, sem).wait()

                def tr_body(g, _):
                    ridx = g * 16 + iota16
                    lov = lo_v[pl.ds(g * 16, 16)]
                    srow = sub * 4 + lax.div(g, 8)
                    l0 = lax.rem(g, 8) * 16
                    for d in range(D):
                        src = plsc.load_gather(rows_v, [ridx, lov + d])
                        xt_v[8 * d + srow, pl.ds(l0, 16)] = src
                    return ()

                lax.fori_loop(0, _GC // 16, tr_body, ())
            for d in range(D):
                pltpu.sync_copy(
                    xt_v.at[pl.ds(8 * d, 8), :],
                    xt_hbm.at[D * f + d, pl.ds(c * 8, 8), :],
                )
            return ()

        lax.fori_loop(0, _UPW, unit_body, ())

    return k(inputs_flat, tables8)


def _sc_shuffle_gate(xt3, perm3, theta_flat):
    mesh = plsc.VectorSubcoreMesh(core_axis_name="c", subcore_axis_name="s")

    @functools.partial(
        pl.kernel,
        out_type=jax.ShapeDtypeStruct((FD, BS, 128), jnp.float32),
        mesh=mesh,
        scratch_types=[
            pltpu.VMEM((BS, 128), jnp.float32),
            pltpu.VMEM((BS, 128), jnp.int32),
            pltpu.VMEM((BS, 128), jnp.float32),
            pltpu.VMEM((FD,), jnp.float32),
            pltpu.SemaphoreType.DMA,
            pltpu.SemaphoreType.DMA,
        ],
        compiler_params=_SC_PARAMS,
    )
    def k(xt_hbm, perm_hbm, th_hbm, comb_hbm, col_v, pidx_v, out_v, th_v,
          sem1, sem2):
        wid = lax.axis_index("s") * _NC + lax.axis_index("c")
        pltpu.sync_copy(th_hbm, th_v)

        def row_body(t, _):
            j = wid * _RPW + t
            cp1 = pltpu.async_copy(xt_hbm.at[j], col_v, sem1)
            cp2 = pltpu.async_copy(perm_hbm.at[j], pidx_v, sem2)
            cp1.wait()
            cp2.wait()
            thj = plsc.load_gather(th_v, [jnp.full((16,), 0, jnp.int32) + j])
            gj = 1.0 / (1.0 + jnp.exp(thj * (-TEMP)))

            def s_body(s, _):
                for l in range(8):
                    pv = pidx_v[s, pl.ds(l * 16, 16)]
                    sidx = lax.shift_right_logical(pv, 7)
                    lidx = lax.bitwise_and(pv, 127)
                    gath = plsc.load_gather(col_v, [sidx, lidx])
                    straight = col_v[s, pl.ds(l * 16, 16)]
                    out_v[s, pl.ds(l * 16, 16)] = gath + gj * (straight - gath)
                return ()

            lax.fori_loop(0, BS, s_body, ())
            pltpu.sync_copy(out_v, comb_hbm.at[j])
            return ()

        lax.fori_loop(0, _RPW, row_body, ())

    return k(xt3, perm3, theta_flat)


def _tc_matmul(comb3, theta_row, weight):
    BM = 1024

    def body(c_ref, th_ref, w_ref, out_ref, loss_ref):
        ct = c_ref[...].reshape(FD, BM)
        out_ref[...] = lax.dot_general(
            ct,
            w_ref[...],
            (((0,), (0,)), ((), ())),
            preferred_element_type=jnp.float32,
        )

        @pl.when(pl.program_id(0) == 0)
        def _():
            loss_ref[0, 0] = jnp.mean(jax.nn.sigmoid(th_ref[...] * TEMP))

    out, loss = pl.pallas_call(
        body,
        grid=(B // BM,),
        in_specs=[
            pl.BlockSpec((FD, BM // 128, 128), lambda i: (0, i, 0)),
            pl.BlockSpec((1, FD), lambda i: (0, 0)),
            pl.BlockSpec((FD, ADAPT), lambda i: (0, 0)),
        ],
        out_specs=[
            pl.BlockSpec((BM, ADAPT), lambda i: (i, 0)),
            pl.BlockSpec(memory_space=pltpu.SMEM),
        ],
        out_shape=[
            jax.ShapeDtypeStruct((B, ADAPT), jnp.float32),
            jax.ShapeDtypeStruct((1, 1), jnp.float32),
        ],
    )(comb3, theta_row, weight)
    return out, loss[0, 0]


def kernel(inputs, tables, theta, weight):
    inputs_flat = inputs.T.reshape(F * B)
    tables8 = _sc_relayout(tables).reshape(F * V // 8, 8 * D)
    xt3 = _sc_gather_t(inputs_flat, tables8)
    comb3 = _sc_shuffle_gate(xt3, _perm3(), theta.reshape(FD))
    out, loss = _tc_matmul(comb3, theta.reshape(1, FD), weight)
    return out, loss


# speedup vs baseline: 1.8495x; 1.8495x over previous
import functools

import jax
import jax.numpy as jnp
from jax import lax
from jax.experimental import pallas as pl
from jax.experimental.pallas import tpu as pltpu
from jax.experimental.pallas import tpu_sc as plsc

F = 26
V = 100000
D = 16
B = 16384
ADAPT = 64
TEMP = 5.0
FD = F * D
BS = B // 128

_NC = 2
_NS = 16
_NW = _NC * _NS
_BC = 1024
_NCH = B // _BC
_UPW = F * _NCH // _NW
_RPW = FD // _NW

_SC_PARAMS = pltpu.CompilerParams(needs_layout_passes=False)
_GC = 512


@functools.cache
def _perm3():
    def build():
        u = jax.random.uniform(jax.random.key(1), (FD, B))
        p = jnp.argsort(u, axis=1).astype(jnp.int32)
        return p.reshape(FD, BS, 128)

    try:
        with jax.ensure_compile_time_eval():
            return build()
    except Exception:
        return build()


def _sc_gather_t(inputs_flat, tables8):
    mesh = plsc.VectorSubcoreMesh(core_axis_name="c", subcore_axis_name="s")

    @functools.partial(
        pl.kernel,
        out_type=jax.ShapeDtypeStruct((FD, BS, 128), jnp.float32),
        mesh=mesh,
        scratch_types=[
            pltpu.VMEM((_GC,), jnp.int32),
            pltpu.VMEM((_GC,), jnp.int32),
            pltpu.VMEM((_GC,), jnp.int32),
            pltpu.VMEM((_GC, 128), jnp.float32),
            pltpu.VMEM((D * 8, 128), jnp.float32),
            pltpu.SemaphoreType.DMA,
        ],
        compiler_params=_SC_PARAMS,
    )
    def k(inp_hbm, tab_hbm, xt_hbm, raw_v, hi_v, lo_v, rows_v, xt_v, sem):
        wid = lax.axis_index("s") * _NC + lax.axis_index("c")
        iota16 = lax.iota(jnp.int32, 16)

        def unit_body(t, _):
            u = wid * _UPW + t
            f = u // _NCH
            c = u % _NCH
            for sub in range(_BC // _GC):
                base = f * B + c * _BC + sub * _GC
                pltpu.sync_copy(inp_hbm.at[pl.ds(base, _GC)], raw_v)

                def idx_body(i, _):
                    raw = raw_v[pl.ds(i * 16, 16)]
                    hi_v[pl.ds(i * 16, 16)] = lax.shift_right_logical(raw, 3)
                    lo_v[pl.ds(i * 16, 16)] = lax.shift_left(
                        lax.bitwise_and(raw, 7), 4
                    )
                    return ()

                lax.fori_loop(0, _GC // 16, idx_body, (), unroll=4)
                pltpu.async_copy(tab_hbm.at[f].at[hi_v], rows_v, sem).wait()

                def tr_body(g, _):
                    ridx = g * 16 + iota16
                    lov = lo_v[pl.ds(g * 16, 16)]
                    srow = sub * 4 + lax.div(g, 8)
                    l0 = lax.rem(g, 8) * 16
                    for d in range(D):
                        src = plsc.load_gather(rows_v, [ridx, lov + d])
                        xt_v[8 * d + srow, pl.ds(l0, 16)] = src
                    return ()

                lax.fori_loop(0, _GC // 16, tr_body, ())
            for d in range(D):
                pltpu.sync_copy(
                    xt_v.at[pl.ds(8 * d, 8), :],
                    xt_hbm.at[D * f + d, pl.ds(c * 8, 8), :],
                )
            return ()

        lax.fori_loop(0, _UPW, unit_body, ())

    return k(inputs_flat, tables8)


def _sc_shuffle_gate(xt3, perm3, theta_flat):
    mesh = plsc.VectorSubcoreMesh(core_axis_name="c", subcore_axis_name="s")

    @functools.partial(
        pl.kernel,
        out_type=jax.ShapeDtypeStruct((FD, BS, 128), jnp.float32),
        mesh=mesh,
        scratch_types=[
            pltpu.VMEM((BS, 128), jnp.float32),
            pltpu.VMEM((BS, 128), jnp.int32),
            pltpu.VMEM((BS, 128), jnp.float32),
            pltpu.VMEM((FD,), jnp.float32),
            pltpu.SemaphoreType.DMA,
            pltpu.SemaphoreType.DMA,
        ],
        compiler_params=_SC_PARAMS,
    )
    def k(xt_hbm, perm_hbm, th_hbm, comb_hbm, col_v, pidx_v, out_v, th_v,
          sem1, sem2):
        wid = lax.axis_index("s") * _NC + lax.axis_index("c")
        pltpu.sync_copy(th_hbm, th_v)

        def row_body(t, _):
            j = wid * _RPW + t
            cp1 = pltpu.async_copy(xt_hbm.at[j], col_v, sem1)
            cp2 = pltpu.async_copy(perm_hbm.at[j], pidx_v, sem2)
            cp1.wait()
            cp2.wait()
            thj = plsc.load_gather(th_v, [jnp.full((16,), 0, jnp.int32) + j])
            gj = 1.0 / (1.0 + jnp.exp(thj * (-TEMP)))

            def s_body(s, _):
                for l in range(8):
                    pv = pidx_v[s, pl.ds(l * 16, 16)]
                    sidx = lax.shift_right_logical(pv, 7)
                    lidx = lax.bitwise_and(pv, 127)
                    gath = plsc.load_gather(col_v, [sidx, lidx])
                    straight = col_v[s, pl.ds(l * 16, 16)]
                    out_v[s, pl.ds(l * 16, 16)] = gath + gj * (straight - gath)
                return ()

            lax.fori_loop(0, BS, s_body, ())
            pltpu.sync_copy(out_v, comb_hbm.at[j])
            return ()

        lax.fori_loop(0, _RPW, row_body, ())

    return k(xt3, perm3, theta_flat)


def _tc_matmul(comb3, theta_row, weight):
    BM = 1024

    def body(c_ref, th_ref, w_ref, out_ref, loss_ref):
        ct = c_ref[...].reshape(FD, BM)
        out_ref[...] = lax.dot_general(
            ct,
            w_ref[...],
            (((0,), (0,)), ((), ())),
            preferred_element_type=jnp.float32,
        )

        @pl.when(pl.program_id(0) == 0)
        def _():
            loss_ref[0, 0] = jnp.mean(jax.nn.sigmoid(th_ref[...] * TEMP))

    out, loss = pl.pallas_call(
        body,
        grid=(B // BM,),
        in_specs=[
            pl.BlockSpec((FD, BM // 128, 128), lambda i: (0, i, 0)),
            pl.BlockSpec((1, FD), lambda i: (0, 0)),
            pl.BlockSpec((FD, ADAPT), lambda i: (0, 0)),
        ],
        out_specs=[
            pl.BlockSpec((BM, ADAPT), lambda i: (i, 0)),
            pl.BlockSpec(memory_space=pltpu.SMEM),
        ],
        out_shape=[
            jax.ShapeDtypeStruct((B, ADAPT), jnp.float32),
            jax.ShapeDtypeStruct((1, 1), jnp.float32),
        ],
    )(comb3, theta_row, weight)
    return out, loss[0, 0]


def kernel(inputs, tables, theta, weight):
    inputs_flat = inputs.T.reshape(F * B)
    tables8 = tables.reshape(F, V // 8, 8 * D)
    xt3 = _sc_gather_t(inputs_flat, tables8)
    comb3 = _sc_shuffle_gate(xt3, _perm3(), theta.reshape(FD))
    out, loss = _tc_matmul(comb3, theta.reshape(1, FD), weight)
    return out, loss


# speedup vs baseline: 1.9054x; 1.0302x over previous
import functools

import jax
import jax.numpy as jnp
from jax import lax
from jax.experimental import pallas as pl
from jax.experimental.pallas import tpu as pltpu
from jax.experimental.pallas import tpu_sc as plsc

F = 26
V = 100000
D = 16
B = 16384
ADAPT = 64
TEMP = 5.0
FD = F * D
BS = B // 128

_NC = 2
_NS = 16
_NW = _NC * _NS
_BC = 1024
_NCH = B // _BC
_UPW = F * _NCH // _NW
_RPW = FD // _NW

_SC_PARAMS = pltpu.CompilerParams(needs_layout_passes=False)
_GC = 512


@functools.cache
def _perm3():
    def build():
        u = jax.random.uniform(jax.random.key(1), (FD, B))
        p = jnp.argsort(u, axis=1).astype(jnp.int32)
        return p.reshape(FD, BS, 128)

    try:
        with jax.ensure_compile_time_eval():
            return build()
    except Exception:
        return build()


def _sc_gather_t(inputs_flat, tables8):
    mesh = plsc.VectorSubcoreMesh(core_axis_name="c", subcore_axis_name="s")

    @functools.partial(
        pl.kernel,
        out_type=jax.ShapeDtypeStruct((FD, BS, 128), jnp.float32),
        mesh=mesh,
        scratch_types=[
            pltpu.VMEM((_GC,), jnp.int32),
            pltpu.VMEM((_GC,), jnp.int32),
            pltpu.VMEM((_GC,), jnp.int32),
            pltpu.VMEM((_GC, 128), jnp.float32),
            pltpu.VMEM((D * 8, 128), jnp.float32),
            pltpu.SemaphoreType.DMA,
        ],
        compiler_params=_SC_PARAMS,
    )
    def k(inp_hbm, tab_hbm, xt_hbm, raw_v, hi_v, lo_v, rows_v, xt_v, sem):
        wid = lax.axis_index("s") * _NC + lax.axis_index("c")
        iota16 = lax.iota(jnp.int32, 16)

        def unit_body(t, _):
            u = wid * _UPW + t
            f = u // _NCH
            c = u % _NCH
            for sub in range(_BC // _GC):
                base = f * B + c * _BC + sub * _GC
                pltpu.sync_copy(inp_hbm.at[pl.ds(base, _GC)], raw_v)
                off = jnp.full((16,), 0, jnp.int32) + f * (V // 8)

                def idx_body(i, _):
                    raw = raw_v[pl.ds(i * 16, 16)]
                    hi_v[pl.ds(i * 16, 16)] = (
                        lax.shift_right_logical(raw, 3) + off
                    )
                    lo_v[pl.ds(i * 16, 16)] = lax.shift_left(
                        lax.bitwise_and(raw, 7), 4
                    )
                    return ()

                lax.fori_loop(0, _GC // 16, idx_body, (), unroll=4)
                pltpu.async_copy(tab_hbm.at[hi_v], rows_v, sem).wait()

                def tr_body(g, _):
                    ridx = g * 16 + iota16
                    lov = lo_v[pl.ds(g * 16, 16)]
                    srow = sub * 4 + lax.div(g, 8)
                    l0 = lax.rem(g, 8) * 16
                    for d in range(D):
                        src = plsc.load_gather(rows_v, [ridx, lov + d])
                        xt_v[8 * d + srow, pl.ds(l0, 16)] = src
                    return ()

                lax.fori_loop(0, _GC // 16, tr_body, ())
            for d in range(D):
                pltpu.sync_copy(
                    xt_v.at[pl.ds(8 * d, 8), :],
                    xt_hbm.at[D * f + d, pl.ds(c * 8, 8), :],
                )
            return ()

        lax.fori_loop(0, _UPW, unit_body, ())

    return k(inputs_flat, tables8)


def _sc_shuffle_gate(xt3, perm3, theta_flat):
    mesh = plsc.VectorSubcoreMesh(core_axis_name="c", subcore_axis_name="s")

    @functools.partial(
        pl.kernel,
        out_type=jax.ShapeDtypeStruct((FD, BS, 128), jnp.float32),
        mesh=mesh,
        scratch_types=[
            pltpu.VMEM((BS, 128), jnp.float32),
            pltpu.VMEM((BS, 128), jnp.int32),
            pltpu.VMEM((BS, 128), jnp.float32),
            pltpu.VMEM((FD,), jnp.float32),
            pltpu.SemaphoreType.DMA,
            pltpu.SemaphoreType.DMA,
        ],
        compiler_params=_SC_PARAMS,
    )
    def k(xt_hbm, perm_hbm, th_hbm, comb_hbm, col_v, pidx_v, out_v, th_v,
          sem1, sem2):
        wid = lax.axis_index("s") * _NC + lax.axis_index("c")
        pltpu.sync_copy(th_hbm, th_v)

        def row_body(t, _):
            j = wid * _RPW + t
            cp1 = pltpu.async_copy(xt_hbm.at[j], col_v, sem1)
            cp2 = pltpu.async_copy(perm_hbm.at[j], pidx_v, sem2)
            cp1.wait()
            cp2.wait()
            thj = plsc.load_gather(th_v, [jnp.full((16,), 0, jnp.int32) + j])
            gj = 1.0 / (1.0 + jnp.exp(thj * (-TEMP)))

            def s_body(s, _):
                for l in range(8):
                    pv = pidx_v[s, pl.ds(l * 16, 16)]
                    sidx = lax.shift_right_logical(pv, 7)
                    lidx = lax.bitwise_and(pv, 127)
                    gath = plsc.load_gather(col_v, [sidx, lidx])
                    straight = col_v[s, pl.ds(l * 16, 16)]
                    out_v[s, pl.ds(l * 16, 16)] = gath + gj * (straight - gath)
                return ()

            lax.fori_loop(0, BS, s_body, ())
            pltpu.sync_copy(out_v, comb_hbm.at[j])
            return ()

        lax.fori_loop(0, _RPW, row_body, ())

    return k(xt3, perm3, theta_flat)


def _tc_matmul(comb3, theta_row, weight):
    BM = 1024

    def body(c_ref, th_ref, w_ref, out_ref, loss_ref):
        ct = c_ref[...].reshape(FD, BM)
        out_ref[...] = lax.dot_general(
            ct,
            w_ref[...],
            (((0,), (0,)), ((), ())),
            preferred_element_type=jnp.float32,
        )

        @pl.when(pl.program_id(0) == 0)
        def _():
            loss_ref[0, 0] = jnp.mean(jax.nn.sigmoid(th_ref[...] * TEMP))

    out, loss = pl.pallas_call(
        body,
        grid=(B // BM,),
        in_specs=[
            pl.BlockSpec((FD, BM // 128, 128), lambda i: (0, i, 0)),
            pl.BlockSpec((1, FD), lambda i: (0, 0)),
            pl.BlockSpec((FD, ADAPT), lambda i: (0, 0)),
        ],
        out_specs=[
            pl.BlockSpec((BM, ADAPT), lambda i: (i, 0)),
            pl.BlockSpec(memory_space=pltpu.SMEM),
        ],
        out_shape=[
            jax.ShapeDtypeStruct((B, ADAPT), jnp.float32),
            jax.ShapeDtypeStruct((1, 1), jnp.float32),
        ],
    )(comb3, theta_row, weight)
    return out, loss[0, 0]


def kernel(inputs, tables, theta, weight):
    inputs_flat = inputs.T.reshape(F * B)
    tables8 = tables.reshape(F * V // 8, 8 * D)
    xt3 = _sc_gather_t(inputs_flat, tables8)
    comb3 = _sc_shuffle_gate(xt3, _perm3(), theta.reshape(FD))
    out, loss = _tc_matmul(comb3, theta.reshape(1, FD), weight)
    return out, loss


# speedup vs baseline: 1.9189x; 1.0070x over previous
import functools

import jax
import jax.numpy as jnp
from jax import lax
from jax.experimental import pallas as pl
from jax.experimental.pallas import tpu as pltpu
from jax.experimental.pallas import tpu_sc as plsc

F = 26
V = 100000
D = 16
B = 16384
ADAPT = 64
TEMP = 5.0
FD = F * D
BS = B // 128

_NC = 2
_NS = 16
_NW = _NC * _NS
_BC = 1024
_NCH = B // _BC
_UPW = F * _NCH // _NW
_RPW = FD // _NW

_SC_PARAMS = pltpu.CompilerParams(needs_layout_passes=False)
_GC = 512


@functools.cache
def _perm3():
    def build():
        u = jax.random.uniform(jax.random.key(1), (FD, B))
        p = jnp.argsort(u, axis=1).astype(jnp.int32)
        return p.reshape(FD, BS, 128)

    try:
        with jax.ensure_compile_time_eval():
            return build()
    except Exception:
        return build()


def _sc_gather_t(inputs_flat, tables8):
    mesh = plsc.VectorSubcoreMesh(core_axis_name="c", subcore_axis_name="s")

    @functools.partial(
        pl.kernel,
        out_type=jax.ShapeDtypeStruct((FD, BS, 128), jnp.float32),
        mesh=mesh,
        scratch_types=[
            pltpu.VMEM((_GC,), jnp.int32),
            pltpu.VMEM((_GC,), jnp.int32),
            pltpu.VMEM((_GC,), jnp.int32),
            pltpu.VMEM((_GC, 128), jnp.float32),
            pltpu.VMEM((D * 8, 128), jnp.float32),
            pltpu.SemaphoreType.DMA,
        ],
        compiler_params=_SC_PARAMS,
    )
    def k(inp_hbm, tab_hbm, xt_hbm, raw_v, hi_v, lo_v, rows_v, xt_v, sem):
        wid = lax.axis_index("s") * _NC + lax.axis_index("c")
        iota16 = lax.iota(jnp.int32, 16)

        def unit_body(t, _):
            u = wid * _UPW + t
            f = u // _NCH
            c = u % _NCH
            for sub in range(_BC // _GC):
                base = f * B + c * _BC + sub * _GC
                pltpu.sync_copy(inp_hbm.at[pl.ds(base, _GC)], raw_v)
                off = jnp.full((16,), 0, jnp.int32) + f * (V // 8)

                def idx_body(i, _):
                    raw = raw_v[pl.ds(i * 16, 16)]
                    hi_v[pl.ds(i * 16, 16)] = (
                        lax.shift_right_logical(raw, 3) + off
                    )
                    lo_v[pl.ds(i * 16, 16)] = lax.shift_left(
                        lax.bitwise_and(raw, 7), 4
                    )
                    return ()

                lax.fori_loop(0, _GC // 16, idx_body, (), unroll=4)
                pltpu.async_copy(tab_hbm.at[hi_v], rows_v, sem).wait()

                def tr_body(g, _):
                    ridx = g * 16 + iota16
                    lov = lo_v[pl.ds(g * 16, 16)]
                    srow = sub * 4 + lax.div(g, 8)
                    l0 = lax.rem(g, 8) * 16
                    for d in range(D):
                        src = plsc.load_gather(rows_v, [ridx, lov + d])
                        xt_v[8 * d + srow, pl.ds(l0, 16)] = src
                    return ()

                lax.fori_loop(0, _GC // 16, tr_body, ())
            cps = [
                pltpu.async_copy(
                    xt_v.at[pl.ds(8 * d, 8), :],
                    xt_hbm.at[D * f + d, pl.ds(c * 8, 8), :],
                    sem,
                )
                for d in range(D)
            ]
            for cp in cps:
                cp.wait()
            return ()

        lax.fori_loop(0, _UPW, unit_body, ())

    return k(inputs_flat, tables8)


def _sc_shuffle_gate(xt3, perm3, theta_flat):
    mesh = plsc.VectorSubcoreMesh(core_axis_name="c", subcore_axis_name="s")

    @functools.partial(
        pl.kernel,
        out_type=jax.ShapeDtypeStruct((FD, BS, 128), jnp.float32),
        mesh=mesh,
        scratch_types=[
            pltpu.VMEM((BS, 128), jnp.float32),
            pltpu.VMEM((BS, 128), jnp.int32),
            pltpu.VMEM((BS, 128), jnp.float32),
            pltpu.VMEM((FD,), jnp.float32),
            pltpu.SemaphoreType.DMA,
            pltpu.SemaphoreType.DMA,
        ],
        compiler_params=_SC_PARAMS,
    )
    def k(xt_hbm, perm_hbm, th_hbm, comb_hbm, col_v, pidx_v, out_v, th_v,
          sem1, sem2):
        wid = lax.axis_index("s") * _NC + lax.axis_index("c")
        pltpu.sync_copy(th_hbm, th_v)

        def row_body(t, _):
            j = wid * _RPW + t
            cp1 = pltpu.async_copy(xt_hbm.at[j], col_v, sem1)
            cp2 = pltpu.async_copy(perm_hbm.at[j], pidx_v, sem2)
            cp1.wait()
            cp2.wait()
            thj = plsc.load_gather(th_v, [jnp.full((16,), 0, jnp.int32) + j])
            gj = 1.0 / (1.0 + jnp.exp(thj * (-TEMP)))

            def s_body(s, _):
                for l in range(8):
                    pv = pidx_v[s, pl.ds(l * 16, 16)]
                    sidx = lax.shift_right_logical(pv, 7)
                    lidx = lax.bitwise_and(pv, 127)
                    gath = plsc.load_gather(col_v, [sidx, lidx])
                    straight = col_v[s, pl.ds(l * 16, 16)]
                    out_v[s, pl.ds(l * 16, 16)] = gath + gj * (straight - gath)
                return ()

            lax.fori_loop(0, BS, s_body, ())
            pltpu.sync_copy(out_v, comb_hbm.at[j])
            return ()

        lax.fori_loop(0, _RPW, row_body, ())

    return k(xt3, perm3, theta_flat)


def _tc_matmul(comb3, theta_row, weight):
    BM = 1024

    def body(c_ref, th_ref, w_ref, out_ref, loss_ref):
        ct = c_ref[...].reshape(FD, BM)
        out_ref[...] = lax.dot_general(
            ct,
            w_ref[...],
            (((0,), (0,)), ((), ())),
            preferred_element_type=jnp.float32,
        )

        @pl.when(pl.program_id(0) == 0)
        def _():
            loss_ref[0, 0] = jnp.mean(jax.nn.sigmoid(th_ref[...] * TEMP))

    out, loss = pl.pallas_call(
        body,
        grid=(B // BM,),
        in_specs=[
            pl.BlockSpec((FD, BM // 128, 128), lambda i: (0, i, 0)),
            pl.BlockSpec((1, FD), lambda i: (0, 0)),
            pl.BlockSpec((FD, ADAPT), lambda i: (0, 0)),
        ],
        out_specs=[
            pl.BlockSpec((BM, ADAPT), lambda i: (i, 0)),
            pl.BlockSpec(memory_space=pltpu.SMEM),
        ],
        out_shape=[
            jax.ShapeDtypeStruct((B, ADAPT), jnp.float32),
            jax.ShapeDtypeStruct((1, 1), jnp.float32),
        ],
    )(comb3, theta_row, weight)
    return out, loss[0, 0]


def kernel(inputs, tables, theta, weight):
    inputs_flat = inputs.T.reshape(F * B)
    tables8 = tables.reshape(F * V // 8, 8 * D)
    xt3 = _sc_gather_t(inputs_flat, tables8)
    comb3 = _sc_shuffle_gate(xt3, _perm3(), theta.reshape(FD))
    out, loss = _tc_matmul(comb3, theta.reshape(1, FD), weight)
    return out, loss
